# baseline (speedup 1.0000x reference)
import functools

import jax
import jax.numpy as jnp
from jax import lax
from jax.experimental import pallas as pl
from jax.experimental.pallas import tpu as pltpu
from jax.experimental.pallas import tpu_sc as plsc

NC, NS, L = 2, 16, 16
NW = NC * NS
B, T, H = 4096, 200, 64
HP = 128
VOCAB = 1000000

GW = 256
NGRP = (VOCAB - 64) // GW
NI_A = 124
TAIL0 = NGRP * GW

SEQ_PER_W = B // NW
CH = 2
NIT = SEQ_PER_W // CH
ROWS = CH * T
NBUF = 4


def _relayout_body(tblT_hbm, out_hbm, src_v, dst_v, tail_v, tdst_v, *sems):
    rsems, wsems = sems[:2], sems[2:4]
    wid = lax.axis_index("s") * NC + lax.axis_index("c")
    iota = lax.iota(jnp.int32, L)
    rows_c = [iota + cc * L for cc in range(H // L)]

    def start_read(g, b):
        pltpu.async_copy(tblT_hbm.at[:, pl.ds(g * GW, GW)], src_v.at[b], rsems[b])

    def transpose_group(b):
        @pl.loop(0, GW, unroll=4)
        def _col(vv):
            vvec = jnp.full((L,), 0, jnp.int32) + vv
            for cc in range(H // L):
                x = plsc.load_gather(src_v.at[b], [rows_c[cc], vvec])
                dst_v[b, pl.ds(vv * H + cc * L, L)] = x

    g0 = wid
    start_read(g0, 0)

    @pl.loop(0, NI_A, step=2)
    def _grp(i):
        for bb in range(2):
            ii = i + bb
            g = wid + ii * NW

            @pl.when(g < NGRP)
            def _work():
                pltpu.make_async_copy(
                    tblT_hbm.at[:, pl.ds(0, GW)], src_v.at[bb], rsems[bb]
                ).wait()
                gn = wid + (ii + 1) * NW

                @pl.when(gn < NGRP)
                def _pref():
                    start_read(gn, 1 - bb)

                @pl.when(ii >= 2)
                def _drainw():
                    pltpu.make_async_copy(
                        dst_v.at[bb], out_hbm.at[pl.ds(0, GW * H)], wsems[bb]
                    ).wait()

                transpose_group(bb)
                pltpu.async_copy(
                    dst_v.at[bb], out_hbm.at[pl.ds(g * GW * H, GW * H)], wsems[bb]
                )

    for bb in range(2):
        @pl.when((wid + (NI_A - 2 + bb) * NW < NGRP) | (wid + (NI_A - 4 + bb) * NW < NGRP))
        def _fin():
            pltpu.make_async_copy(
                dst_v.at[bb], out_hbm.at[pl.ds(0, GW * H)], wsems[bb]
            ).wait()

    @pl.when(wid == 0)
    def _tail():
        pltpu.sync_copy(tblT_hbm.at[:, pl.ds(TAIL0, 64)], tail_v)

        @pl.loop(0, 64)
        def _col(vv):
            vvec = jnp.full((L,), 0, jnp.int32) + vv
            for cc in range(H // L):
                x = plsc.load_gather(tail_v, [rows_c[cc], vvec])
                tdst_v[pl.ds(vv * H + cc * L, L)] = x

        pltpu.sync_copy(tdst_v, out_hbm.at[pl.ds(TAIL0 * H, 64 * H)])


def _gather_body(ids_hbm, tok_hbm, pos_hbm, out_hbm, idx_v, rows_v, pos_v, *sems):
    gsems, wsems = sems[:NBUF], sems[NBUF:]
    wid = lax.axis_index("s") * NC + lax.axis_index("c")
    pltpu.sync_copy(pos_hbm, pos_v)
    row_base = wid * SEQ_PER_W * T

    def start_gather(c, b):
        row0 = row_base + c * ROWS
        pltpu.sync_copy(ids_hbm.at[pl.ds(row0, ROWS)], idx_v.at[b])
        pltpu.async_copy(tok_hbm.at[idx_v.at[b]], rows_v.at[b], gsems[b])

    for b in range(2):
        start_gather(b, b)

    @pl.loop(0, NIT, step=NBUF)
    def _grp(g):
        for b in range(NBUF):
            c = g + b
            pltpu.make_async_copy(
                tok_hbm.at[idx_v.at[b]], rows_v.at[b], gsems[b]
            ).wait()

            nb = (b + 2) % NBUF
            nxt = c + 2

            @pl.when(nxt < NIT)
            def _prefetch():
                @pl.when(c >= 2)
                def _drain():
                    pltpu.make_async_copy(
                        rows_v.at[nb],
                        out_hbm.at[pl.ds(0, ROWS), pl.ds(0, H)],
                        wsems[nb],
                    ).wait()

                start_gather(nxt, nb)

            @pl.loop(0, ROWS, unroll=2)
            def _row(r):
                t = lax.rem(r, T)
                for cc in range(H // L):
                    sl = pl.ds(cc * L, L)
                    rows_v[b, r, sl] = rows_v[b, r, sl] + pos_v[t, sl]

            row0 = row_base + c * ROWS
            pltpu.async_copy(
                rows_v.at[b],
                out_hbm.at[pl.ds(row0, ROWS), pl.ds(0, H)],
                wsems[b],
            )

    for b in range(NBUF):
        pltpu.make_async_copy(
            rows_v.at[b], out_hbm.at[pl.ds(0, ROWS), pl.ds(0, H)], wsems[b]
        ).wait()


def _mesh():
    return plsc.VectorSubcoreMesh(
        core_axis_name="c", subcore_axis_name="s", num_cores=NC, num_subcores=NS
    )


@jax.jit
def _run(input_ids, token_emb, pos_emb):
    relayout = pl.kernel(
        _relayout_body,
        out_type=jax.ShapeDtypeStruct((VOCAB * H,), jnp.float32),
        mesh=_mesh(),
        compiler_params=pltpu.CompilerParams(
            use_tc_tiling_on_sc=True, needs_layout_passes=False
        ),
        scratch_types=[
            pltpu.VMEM((2, H, GW), jnp.float32),
            pltpu.VMEM((2, GW * H), jnp.float32),
            pltpu.VMEM((H, 64), jnp.float32),
            pltpu.VMEM((64 * H,), jnp.float32),
        ]
        + [pltpu.SemaphoreType.DMA] * 4,
    )
    gather = pl.kernel(
        _gather_body,
        out_type=jax.ShapeDtypeStruct((B * T, HP), jnp.float32),
        mesh=_mesh(),
        compiler_params=pltpu.CompilerParams(use_tc_tiling_on_sc=False),
        scratch_types=[
            pltpu.VMEM((NBUF, ROWS), jnp.int32),
            pltpu.VMEM((NBUF, ROWS, H), jnp.float32),
            pltpu.VMEM((T, H), jnp.float32),
        ]
        + [pltpu.SemaphoreType.DMA] * (2 * NBUF),
    )
    ids_flat = input_ids.reshape(B * T).astype(jnp.int32)
    tbl_lin = relayout(token_emb.T)
    out = gather(ids_flat, tbl_lin.reshape(VOCAB, H), pos_emb)
    return out.reshape(B, T, HP)[:, :, :H]


def kernel(input_ids, token_emb, pos_emb):
    return _run(input_ids, token_emb, pos_emb)


# speedup vs baseline: 1.1057x; 1.1057x over previous
import functools

import jax
import jax.numpy as jnp
from jax import lax
from jax.experimental import pallas as pl
from jax.experimental.pallas import tpu as pltpu
from jax.experimental.pallas import tpu_sc as plsc

NC, NS, L = 2, 16, 16
NW = NC * NS
B, T, H = 4096, 200, 64
HP = 128
VOCAB = 1000000

BK = 512
GRID = (VOCAB + BK - 1) // BK

SEQ_PER_W = B // NW
CH = 1
NIT = SEQ_PER_W // CH
ROWS = CH * T
NBUF = 4


def _transpose_body(x_ref, o_ref):
    t = x_ref[...].T.reshape(BK // 2, 2, H)
    o_ref[:, 0:H] = t[:, 0, :]
    o_ref[:, H:HP] = t[:, 1, :]


@jax.jit
def _run(input_ids, token_emb, pos_emb):
    tbl_lin = pl.pallas_call(
        _transpose_body,
        grid=(GRID,),
        in_specs=[pl.BlockSpec((H, BK), lambda i: (0, i))],
        out_specs=pl.BlockSpec((BK // 2, HP), lambda i: (i, 0)),
        out_shape=jax.ShapeDtypeStruct((VOCAB * H // HP, HP), jnp.float32),
    )(token_emb.T)

    mesh = plsc.VectorSubcoreMesh(
        core_axis_name="c", subcore_axis_name="s", num_cores=NC, num_subcores=NS
    )
    gather = pl.kernel(
        _gather_body,
        out_type=jax.ShapeDtypeStruct((B * T, HP), jnp.float32),
        mesh=mesh,
        compiler_params=pltpu.CompilerParams(use_tc_tiling_on_sc=False),
        scratch_types=[
            pltpu.VMEM((NBUF, ROWS), jnp.int32),
            pltpu.VMEM((NBUF, ROWS, H), jnp.float32),
            pltpu.VMEM((2, ROWS, HP), jnp.float32),
            pltpu.VMEM((T, H), jnp.float32),
        ]
        + [pltpu.SemaphoreType.DMA] * (NBUF + 2),
    )
    ids_flat = input_ids.reshape(B * T).astype(jnp.int32)
    out = gather(ids_flat, tbl_lin.reshape(VOCAB, H), pos_emb)
    return out.reshape(B, T, HP)[:, :, :H]


def _gather_body(ids_hbm, tok_hbm, pos_hbm, out_hbm, idx_v, rows_v, obuf_v, pos_v, *sems):
    gsems = sems[:NBUF]
    wsems = sems[NBUF : NBUF + 2]
    wid = lax.axis_index("s") * NC + lax.axis_index("c")
    pltpu.sync_copy(pos_hbm, pos_v)
    row_base = wid * SEQ_PER_W * T

    def start_gather(c, b):
        row0 = row_base + c * ROWS
        pltpu.sync_copy(ids_hbm.at[pl.ds(row0, ROWS)], idx_v.at[b])
        pltpu.async_copy(tok_hbm.at[idx_v.at[b]], rows_v.at[b], gsems[b])

    for b in range(2):
        start_gather(b, b)

    @pl.loop(0, NIT, step=NBUF)
    def _grp(g):
        for b in range(NBUF):
            bo = b % 2
            c = g + b
            pltpu.make_async_copy(
                tok_hbm.at[idx_v.at[b]], rows_v.at[b], gsems[b]
            ).wait()

            nb = (b + 2) % NBUF
            nxt = c + 2

            @pl.when(nxt < NIT)
            def _prefetch():
                start_gather(nxt, nb)

            @pl.when(c >= 2)
            def _drain():
                pltpu.make_async_copy(
                    obuf_v.at[bo], out_hbm.at[pl.ds(0, ROWS)], wsems[bo]
                ).wait()

            @pl.loop(0, ROWS, unroll=2)
            def _row(r):
                for cc in range(H // L):
                    sl = pl.ds(cc * L, L)
                    obuf_v[bo, r, sl] = rows_v[b, r, sl] + pos_v[r, sl]

            row0 = row_base + c * ROWS
            pltpu.async_copy(
                obuf_v.at[bo], out_hbm.at[pl.ds(row0, ROWS)], wsems[bo]
            )

    for bo in range(2):
        pltpu.make_async_copy(
            obuf_v.at[bo], out_hbm.at[pl.ds(0, ROWS)], wsems[bo]
        ).wait()


def kernel(input_ids, token_emb, pos_emb):
    return _run(input_ids, token_emb, pos_emb)


# speedup vs baseline: 1.9034x; 1.7214x over previous
import functools

import jax
import jax.numpy as jnp
from jax import lax
from jax.experimental import pallas as pl
from jax.experimental.pallas import tpu as pltpu
from jax.experimental.pallas import tpu_sc as plsc

NC, NS, L = 2, 16, 16
NW = NC * NS
B, T, H = 4096, 200, 64
HP = 128
SEQ_PER_W = B // NW
CH = 1
NIT = SEQ_PER_W // CH
ROWS = CH * T
NBUF = 3
ROWS_W = SEQ_PER_W * T


def _body(ids_hbm, tok_hbm, pos_hbm, out_hbm, idx_v, rows_v, pos_v, *sems):
    gsems, wsems = sems[:NBUF], sems[NBUF:]
    wid = lax.axis_index("s") * NC + lax.axis_index("c")
    row_base = wid * ROWS_W
    pltpu.sync_copy(ids_hbm.at[pl.ds(row_base, ROWS_W)], idx_v)
    pltpu.sync_copy(pos_hbm, pos_v)

    def start_gather(c, b):
        pltpu.async_copy(
            tok_hbm.at[idx_v.at[pl.ds(c * ROWS, ROWS)]], rows_v.at[b], gsems[b]
        )

    def wait_gather(c, b):
        pltpu.make_async_copy(
            tok_hbm.at[idx_v.at[pl.ds(c * ROWS, ROWS)]], rows_v.at[b], gsems[b]
        ).wait()

    for b in range(2):
        start_gather(b, b)

    @pl.loop(0, NIT + 1, step=NBUF)
    def _grp(g):
        for b in range(NBUF):
            c = g + b

            @pl.when(c < NIT)
            def _chunk():
                wait_gather(c, b)

                @pl.loop(0, T, unroll=2)
                def _row(t):
                    for cc in range(H // L):
                        sl = pl.ds(cc * L, L)
                        rows_v[b, t, sl] = rows_v[b, t, sl] + pos_v[t, sl]

                nb = (b + 2) % NBUF
                nxt = c + 2

                @pl.when(nxt < NIT)
                def _prefetch():
                    @pl.when(c >= 1)
                    def _drain():
                        pltpu.make_async_copy(
                            rows_v.at[nb], out_hbm.at[pl.ds(0, ROWS)], wsems[nb]
                        ).wait()

                    start_gather(nxt, nb)

                row0 = row_base + c * ROWS
                pltpu.async_copy(
                    rows_v.at[b], out_hbm.at[pl.ds(row0, ROWS)], wsems[b]
                )

    for b in range(NBUF):
        pltpu.make_async_copy(
            rows_v.at[b], out_hbm.at[pl.ds(0, ROWS)], wsems[b]
        ).wait()


@jax.jit
def _run(ids_flat, tok_padded, pos_emb):
    mesh = plsc.VectorSubcoreMesh(
        core_axis_name="c", subcore_axis_name="s", num_cores=NC, num_subcores=NS
    )
    k = pl.kernel(
        _body,
        out_type=jax.ShapeDtypeStruct((B * T, HP), jnp.float32),
        mesh=mesh,
        compiler_params=pltpu.CompilerParams(use_tc_tiling_on_sc=False),
        scratch_types=[
            pltpu.VMEM((ROWS_W,), jnp.int32),
            pltpu.VMEM((NBUF, ROWS, HP), jnp.float32),
            pltpu.VMEM((T, H), jnp.float32),
        ]
        + [pltpu.SemaphoreType.DMA] * (2 * NBUF),
    )
    return k(ids_flat, tok_padded, pos_emb)


def kernel(input_ids, token_emb, pos_emb):
    ids_flat = input_ids.reshape(B * T).astype(jnp.int32)
    tok_padded = jnp.pad(token_emb, ((0, 0), (0, HP - H)))
    out = _run(ids_flat, tok_padded, pos_emb)
    return out.reshape(B, T, HP)[:, :, :H]
